# baseline (speedup 1.0000x reference)
import jax
import jax.numpy as jnp
from jax.experimental import pallas as pl
from jax.experimental.pallas import tpu as pltpu


def _body(x_ref, w1_ref, b1_ref, w2_ref, b2_ref, o_ref):
    s = x_ref.shape[1]
    mean = jnp.sum(x_ref[...].astype(jnp.float32), axis=1) * (1.0 / s)
    h = jnp.dot(mean, w1_ref[...], preferred_element_type=jnp.float32)
    h = jnp.maximum(h + b1_ref[...], 0.0)
    logits = jnp.dot(h, w2_ref[...], preferred_element_type=jnp.float32)
    logits = logits + b2_ref[...]
    m = jnp.max(logits, axis=-1, keepdims=True)
    z = logits - m
    o_ref[...] = z - jnp.log(jnp.sum(jnp.exp(z), axis=-1, keepdims=True))


def _largest_divisor_leq(n, cap, step=8):
    best = None
    for d in range(step, min(n, cap) + 1, step):
        if n % d == 0:
            best = d
    return best


def kernel(x, w1, b1, w2, b2):
    B, S, H = x.shape
    L = w2.shape[1]
    b1 = jnp.asarray(b1, jnp.float32).reshape(1, H)
    b2 = jnp.asarray(b2, jnp.float32).reshape(1, L)

    L_pad = -(-L // 128) * 128
    if L_pad != L:
        w2 = jnp.pad(w2, ((0, 0), (0, L_pad - L)))
        b2 = jnp.pad(b2, ((0, 0), (0, L_pad - L)), constant_values=-1e30)

    block_b = _largest_divisor_leq(B, -(-B // 8)) or B
    grid_b = B // block_b

    out = pl.pallas_call(
        _body,
        out_shape=jax.ShapeDtypeStruct((B, L_pad), jnp.float32),
        grid=(grid_b,),
        in_specs=[
            pl.BlockSpec((block_b, S, H), lambda b: (b, 0, 0)),
            pl.BlockSpec((H, H), lambda b: (0, 0)),
            pl.BlockSpec((1, H), lambda b: (0, 0)),
            pl.BlockSpec((H, L_pad), lambda b: (0, 0)),
            pl.BlockSpec((1, L_pad), lambda b: (0, 0)),
        ],
        out_specs=pl.BlockSpec((block_b, L_pad), lambda b: (b, 0)),
        compiler_params=pltpu.CompilerParams(
            dimension_semantics=("parallel",),
            vmem_limit_bytes=60 * 1024 * 1024,
        ),
    )(x, w1, b1, w2, b2)

    return {"preds_length": out[:, :L]}


# speedup vs baseline: 1.0386x; 1.0386x over previous
import jax
import jax.numpy as jnp
from jax.experimental import pallas as pl
from jax.experimental.pallas import tpu as pltpu

_PARTS = 4


def _body(*refs):
    x_parts = refs[:_PARTS]
    w1_ref, b1_ref, w2_ref, b2_ref, o_ref = refs[_PARTS:]
    s = x_parts[0].shape[1]
    mean = jnp.concatenate(
        [jnp.sum(p[...].astype(jnp.float32), axis=1) for p in x_parts], axis=0
    ) * (1.0 / s)
    h = jnp.dot(mean, w1_ref[...], preferred_element_type=jnp.float32)
    h = jnp.maximum(h + b1_ref[...], 0.0)
    logits = jnp.dot(h, w2_ref[...], preferred_element_type=jnp.float32)
    logits = logits + b2_ref[...]
    m = jnp.max(logits, axis=-1, keepdims=True)
    z = logits - m
    o_ref[...] = z - jnp.log(jnp.sum(jnp.exp(z), axis=-1, keepdims=True))


def _largest_divisor_leq(n, cap, step=8):
    best = None
    for d in range(step, min(n, cap) + 1, step):
        if n % d == 0:
            best = d
    return best


def kernel(x, w1, b1, w2, b2):
    B, S, H = x.shape
    L = w2.shape[1]
    b1 = jnp.asarray(b1, jnp.float32).reshape(1, H)
    b2 = jnp.asarray(b2, jnp.float32).reshape(1, L)

    L_pad = -(-L // 128) * 128
    if L_pad != L:
        w2 = jnp.pad(w2, ((0, 0), (0, L_pad - L)))
        b2 = jnp.pad(b2, ((0, 0), (0, L_pad - L)), constant_values=-1e30)

    block_b = _largest_divisor_leq(B, -(-B // 4)) or B
    grid_b = B // block_b
    sub_b = block_b // _PARTS
    assert block_b % _PARTS == 0

    def part_spec(p):
        return pl.BlockSpec((sub_b, S, H), lambda b, p=p: (b * _PARTS + p, 0, 0))

    out = pl.pallas_call(
        _body,
        out_shape=jax.ShapeDtypeStruct((B, L_pad), jnp.float32),
        grid=(grid_b,),
        in_specs=[part_spec(p) for p in range(_PARTS)]
        + [
            pl.BlockSpec((H, H), lambda b: (0, 0)),
            pl.BlockSpec((1, H), lambda b: (0, 0)),
            pl.BlockSpec((H, L_pad), lambda b: (0, 0)),
            pl.BlockSpec((1, L_pad), lambda b: (0, 0)),
        ],
        out_specs=pl.BlockSpec((block_b, L_pad), lambda b: (b, 0)),
        compiler_params=pltpu.CompilerParams(
            dimension_semantics=("parallel",),
            vmem_limit_bytes=60 * 1024 * 1024,
        ),
    )(*([x] * _PARTS), w1, b1, w2, b2)

    return {"preds_length": out[:, :L]}
